# baseline (speedup 1.0000x reference)
import functools

import jax
import jax.numpy as jnp
import numpy as np
from jax import lax
from jax.experimental import pallas as pl
from jax.experimental.pallas import tpu as pltpu
from jax.experimental.pallas import tpu_sc as plsc

MU1, SIGMA1, MU2, SIGMA2 = 5.79, 0.87, 10.68, 1.78
MIN_SEQ_SEP = 5
MAX_DIST = 12.0
TAU_SQ = 0.02 ** 2

NC, NS, LANES = 2, 16, 16
NW = NC * NS

G_N = 4096
G_R0 = -4.0
G_H = 16.0 / G_N


def _g_table():
    xs = G_R0 + G_H * np.arange(G_N + 1, dtype=np.float64)
    gv = (np.exp(-0.5 * ((xs - MU1) / SIGMA1) ** 2)
          + np.exp(-0.5 * ((xs - MU2) / SIGMA2) ** 2))
    val = gv[:-1]
    slope = gv[1:] - gv[:-1]
    s_min, s_max = slope.min(), slope.max()
    s_step = (s_max - s_min) / 255.0
    q = np.clip(np.round((slope - s_min) / s_step), 0, 255).astype(np.uint32)
    vbits = ((val.astype(np.float32).view(np.uint32) + np.uint32(0x80))
             & np.uint32(0xFFFFFF00))
    return (vbits | q).view(np.int32), float(s_min), float(s_step)


_GTAB, _S_MIN, _S_STEP = _g_table()


def _sc_partials(p_full, r_t, j_t, gtab, B, L, K):
    kpw = K // NW
    vecs = L // LANES

    mesh = plsc.VectorSubcoreMesh(
        core_axis_name="c", subcore_axis_name="s",
        num_cores=NC, num_subcores=NS)

    @functools.partial(
        pl.kernel,
        out_type=jax.ShapeDtypeStruct((NW, B, LANES), jnp.float32),
        mesh=mesh,
        compiler_params=pltpu.CompilerParams(needs_layout_passes=False),
        scratch_types=[
            pltpu.VMEM((L,), jnp.float32),
            pltpu.VMEM((L,), jnp.float32),
            pltpu.VMEM((2, kpw, L), jnp.float32),
            pltpu.VMEM((2, kpw, L), jnp.int32),
            pltpu.VMEM((B, LANES), jnp.float32),
            pltpu.VMEM((G_N,), jnp.int32),
            pltpu.SemaphoreType.DMA,
            pltpu.SemaphoreType.DMA,
        ],
    )
    def k(pf_hbm, r_hbm, j_hbm, gtab_hbm, out_hbm,
          table0, table1, rv, jv, accv, gtab_v, sem0, sem1):
        tables = (table0, table1)
        cid = lax.axis_index("c")
        sid = lax.axis_index("s")
        wid = sid * NC + cid
        k0 = wid * kpw
        sems = (sem0, sem1)
        iota_m5 = lax.iota(jnp.int32, LANES) - MIN_SEQ_SEP

        def start_batch(b, slot):
            pltpu.async_copy(pf_hbm.at[b], tables[slot], sems[slot])
            pltpu.async_copy(r_hbm.at[b, pl.ds(k0, kpw)],
                             rv.at[slot], sems[slot])
            pltpu.async_copy(j_hbm.at[b, pl.ds(k0, kpw)],
                             jv.at[slot], sems[slot])

        def wait_slot(slot):
            pltpu.make_async_copy(pf_hbm.at[0], tables[slot],
                                  sems[slot]).wait()
            pltpu.make_async_copy(r_hbm.at[0, pl.ds(0, kpw)],
                                  rv.at[slot], sems[slot]).wait()
            pltpu.make_async_copy(j_hbm.at[0, pl.ds(0, kpw)],
                                  jv.at[slot], sems[slot]).wait()

        def compute_batch(b, slot):
            table = tables[slot]

            @plsc.parallel_loop(
                0, vecs, unroll=4,
                carry=tuple(jnp.zeros((LANES,), jnp.float32)
                            for _ in range(kpw)))
            def accs(v, accs):
                off = pl.multiple_of(v * LANES, LANES)
                l_m5 = off + iota_m5
                p_i = table[pl.ds(off, LANES)]
                new = []
                for kk in range(kpw):
                    jvec = jv[slot, kk, pl.ds(off, LANES)]
                    rvec = rv[slot, kk, pl.ds(off, LANES)]
                    valid = rvec < (MAX_DIST - 0.0001)
                    sep_ok = (jvec - l_m5).astype(jnp.uint32) \
                        > (2 * MIN_SEQ_SEP)
                    mask = jnp.logical_and(valid, sep_ok)
                    u = rvec * (1.0 / G_H) + (-G_R0 / G_H)
                    u = jnp.minimum(jnp.maximum(u, 0.0), G_N - 0.001)
                    idx = u.astype(jnp.int32)
                    frac = u - idx.astype(jnp.float32)
                    wv = plsc.load_gather(gtab_v, [idx])
                    gv = plsc.bitcast(wv & jnp.int32(-256), jnp.float32)
                    gs = (wv & 255).astype(jnp.float32) * _S_STEP + _S_MIN
                    g = gv + frac * gs
                    p_j = plsc.load_gather(table, [jvec])
                    s = (p_i * p_j) * g
                    s = jnp.where(mask, s, 0.0)
                    s2 = s * s
                    new.append(accs[kk] + s * s2 / (s2 + TAU_SQ))
                return tuple(new)

            total = accs[0]
            for kk in range(1, kpw):
                total = total + accs[kk]
            accv[b] = total

        start_batch(0, 0)
        pltpu.sync_copy(gtab_hbm, gtab_v)

        def pair_body(t, _):
            b = 2 * t
            start_batch(b + 1, 1)
            wait_slot(0)
            compute_batch(b, 0)

            @pl.when(b + 2 < B)
            def _():
                start_batch(b + 2, 0)

            wait_slot(1)
            compute_batch(b + 1, 1)
            return 0

        lax.fori_loop(0, B // 2, pair_body, 0)
        pltpu.sync_copy(accv, out_hbm.at[wid])

    return k(p_full, r_t, j_t, gtab)


def kernel(p_ext, R, r, j_idx, lambda_raw):
    del R
    B, L, K = r.shape
    p_full = jnp.pad(p_ext, ((0, 0), (1, 0)))
    r_t = jnp.transpose(r, (0, 2, 1))
    j_t = jnp.transpose(j_idx, (0, 2, 1))
    partials = _sc_partials(p_full, r_t, j_t, jnp.asarray(_GTAB), B, L, K)
    e_sum = partials.sum(axis=(0, 2))
    lambda_hb = jax.nn.softplus(lambda_raw) + 1e-06
    return -lambda_hb * e_sum / float(max(L, 1))


# speedup vs baseline: 1.0763x; 1.0763x over previous
import functools

import jax
import jax.numpy as jnp
import numpy as np
from jax import lax
from jax.experimental import pallas as pl
from jax.experimental.pallas import tpu as pltpu
from jax.experimental.pallas import tpu_sc as plsc

MU1, SIGMA1, MU2, SIGMA2 = 5.79, 0.87, 10.68, 1.78
MIN_SEQ_SEP = 5
MAX_DIST = 12.0
TAU_SQ = 0.02 ** 2

NC, NS, LANES = 2, 16, 16
NW = NC * NS

G_N = 4096
G_H = 1.0 / 256.0
G_ZERO = 4088
G_RC = float(np.float32(MAX_DIST - 0.0001))
G_R0 = G_RC - G_ZERO * G_H
G_C0 = G_ZERO - G_RC / G_H


def _g_tables():
    xs = G_R0 + G_H * np.arange(G_N + 1, dtype=np.float64)
    gv = (np.exp(-0.5 * ((xs - MU1) / SIGMA1) ** 2)
          + np.exp(-0.5 * ((xs - MU2) / SIGMA2) ** 2))
    gval = gv[:-1].astype(np.float32)
    gslope = (gv[1:] - gv[:-1]).astype(np.float32)
    gval[G_ZERO:] = 0.0
    gslope[G_ZERO:] = 0.0
    return gval, gslope


_GVAL, _GSLOPE = _g_tables()


def _sc_partials(p_full, r_t, j_t, gval, gslope, B, L, K):
    kpw = K // NW
    vecs = L // LANES

    mesh = plsc.VectorSubcoreMesh(
        core_axis_name="c", subcore_axis_name="s",
        num_cores=NC, num_subcores=NS)

    @functools.partial(
        pl.kernel,
        out_type=jax.ShapeDtypeStruct((NW, B, LANES), jnp.float32),
        mesh=mesh,
        compiler_params=pltpu.CompilerParams(needs_layout_passes=False),
        scratch_types=[
            pltpu.VMEM((L,), jnp.float32),
            pltpu.VMEM((L,), jnp.float32),
            pltpu.VMEM((2, kpw, L), jnp.float32),
            pltpu.VMEM((2, kpw, L), jnp.int32),
            pltpu.VMEM((B, LANES), jnp.float32),
            pltpu.VMEM((G_N,), jnp.float32),
            pltpu.VMEM((G_N,), jnp.float32),
            pltpu.SemaphoreType.DMA,
            pltpu.SemaphoreType.DMA,
        ],
    )
    def k(pf_hbm, r_hbm, j_hbm, gval_hbm, gslope_hbm, out_hbm,
          table0, table1, rv, jv, accv, gval_v, gslope_v, sem0, sem1):
        tables = (table0, table1)
        cid = lax.axis_index("c")
        sid = lax.axis_index("s")
        wid = sid * NC + cid
        k0 = wid * kpw
        sems = (sem0, sem1)
        iota_m5 = lax.iota(jnp.int32, LANES) - MIN_SEQ_SEP

        def start_batch(b, slot):
            pltpu.async_copy(pf_hbm.at[b], tables[slot], sems[slot])
            pltpu.async_copy(r_hbm.at[b, pl.ds(k0, kpw)],
                             rv.at[slot], sems[slot])
            pltpu.async_copy(j_hbm.at[b, pl.ds(k0, kpw)],
                             jv.at[slot], sems[slot])

        def wait_slot(slot):
            pltpu.make_async_copy(pf_hbm.at[0], tables[slot],
                                  sems[slot]).wait()
            pltpu.make_async_copy(r_hbm.at[0, pl.ds(0, kpw)],
                                  rv.at[slot], sems[slot]).wait()
            pltpu.make_async_copy(j_hbm.at[0, pl.ds(0, kpw)],
                                  jv.at[slot], sems[slot]).wait()

        def compute_batch(b, slot):
            table = tables[slot]

            @plsc.parallel_loop(
                0, L, step=LANES, unroll=4,
                carry=tuple(jnp.zeros((LANES,), jnp.float32)
                            for _ in range(kpw)))
            def accs(v, accs):
                off = pl.multiple_of(v, LANES)
                l_m5 = off + iota_m5
                p_i = table[pl.ds(off, LANES)]
                new = []
                for kk in range(kpw):
                    jvec = jv[slot, kk, pl.ds(off, LANES)]
                    rvec = rv[slot, kk, pl.ds(off, LANES)]
                    sep_ok = (jvec - l_m5).astype(jnp.uint32) \
                        > (2 * MIN_SEQ_SEP)
                    u = rvec * (1.0 / G_H) + G_C0
                    u = jnp.minimum(jnp.maximum(u, 0.0), G_N - 0.001)
                    idx = u.astype(jnp.int32)
                    frac = u - idx.astype(jnp.float32)
                    g = plsc.load_gather(gval_v, [idx]) \
                        + frac * plsc.load_gather(gslope_v, [idx])
                    p_j = plsc.load_gather(table, [jvec])
                    s = (p_i * p_j) * g
                    s = jnp.where(sep_ok, s, 0.0)
                    s2 = s * s
                    new.append(accs[kk] + s * s2 / (s2 + TAU_SQ))
                return tuple(new)

            total = accs[0]
            for kk in range(1, kpw):
                total = total + accs[kk]
            accv[b] = total

        start_batch(0, 0)
        pltpu.sync_copy(gval_hbm, gval_v)
        pltpu.sync_copy(gslope_hbm, gslope_v)

        def pair_body(t, _):
            b = 2 * t
            start_batch(b + 1, 1)
            wait_slot(0)
            compute_batch(b, 0)

            @pl.when(b + 2 < B)
            def _():
                start_batch(b + 2, 0)

            wait_slot(1)
            compute_batch(b + 1, 1)
            return 0

        lax.fori_loop(0, B // 2, pair_body, 0)
        pltpu.sync_copy(accv, out_hbm.at[wid])

    return k(p_full, r_t, j_t, gval, gslope)


def kernel(p_ext, R, r, j_idx, lambda_raw):
    del R
    B, L, K = r.shape
    p_full = jnp.pad(p_ext, ((0, 0), (1, 0)))
    r_t = jnp.transpose(r, (0, 2, 1))
    j_t = jnp.transpose(j_idx, (0, 2, 1))
    partials = _sc_partials(p_full, r_t, j_t,
                            jnp.asarray(_GVAL), jnp.asarray(_GSLOPE),
                            B, L, K)
    e_sum = partials.sum(axis=(0, 2))
    lambda_hb = jax.nn.softplus(lambda_raw) + 1e-06
    return -lambda_hb * e_sum / float(max(L, 1))


# speedup vs baseline: 1.0843x; 1.0074x over previous
import functools

import jax
import jax.numpy as jnp
import numpy as np
from jax import lax
from jax.experimental import pallas as pl
from jax.experimental.pallas import tpu as pltpu
from jax.experimental.pallas import tpu_sc as plsc

MU1, SIGMA1, MU2, SIGMA2 = 5.79, 0.87, 10.68, 1.78
MIN_SEQ_SEP = 5
MAX_DIST = 12.0
TAU_SQ = 0.02 ** 2

NC, NS, LANES = 2, 16, 16
NW = NC * NS

G_N = 4096
G_H = 1.0 / 256.0
G_ZERO = 4088
G_RC = float(np.float32(MAX_DIST - 0.0001))
G_R0 = G_RC - G_ZERO * G_H
G_C0 = G_ZERO - G_RC / G_H


def _g_tables():
    xs = G_R0 + G_H * np.arange(G_N + 1, dtype=np.float64)
    gv = (np.exp(-0.5 * ((xs - MU1) / SIGMA1) ** 2)
          + np.exp(-0.5 * ((xs - MU2) / SIGMA2) ** 2))
    gval = gv[:-1].astype(np.float32)
    gslope = (gv[1:] - gv[:-1]).astype(np.float32)
    gval[G_ZERO:] = 0.0
    gslope[G_ZERO:] = 0.0
    return gval, gslope


_GVAL, _GSLOPE = _g_tables()


def _sc_partials(p_full, r_t, j_t, gval, gslope, B, L, K):
    halves = NW // B
    kph = K // halves
    kpw = 2
    nsub = kph // kpw

    mesh = plsc.VectorSubcoreMesh(
        core_axis_name="c", subcore_axis_name="s",
        num_cores=NC, num_subcores=NS)

    @functools.partial(
        pl.kernel,
        out_type=jax.ShapeDtypeStruct((NW, LANES), jnp.float32),
        mesh=mesh,
        compiler_params=pltpu.CompilerParams(needs_layout_passes=False),
        scratch_types=[
            pltpu.VMEM((L,), jnp.float32),
            pltpu.VMEM((2, kpw, L), jnp.float32),
            pltpu.VMEM((2, kpw, L), jnp.int32),
            pltpu.VMEM((LANES,), jnp.float32),
            pltpu.VMEM((G_N,), jnp.float32),
            pltpu.VMEM((G_N,), jnp.float32),
            pltpu.SemaphoreType.DMA,
            pltpu.SemaphoreType.DMA,
        ],
    )
    def k(pf_hbm, r_hbm, j_hbm, gval_hbm, gslope_hbm, out_hbm,
          table, rv, jv, accv, gval_v, gslope_v, sem0, sem1):
        cid = lax.axis_index("c")
        sid = lax.axis_index("s")
        wid = sid * NC + cid
        b = wid // halves
        k0 = (wid % halves) * kph
        sems = (sem0, sem1)
        iota_m5 = lax.iota(jnp.int32, LANES) - MIN_SEQ_SEP

        def start_sub(sub, slot):
            pltpu.async_copy(r_hbm.at[b, pl.ds(k0 + sub * kpw, kpw)],
                             rv.at[slot], sems[slot])
            pltpu.async_copy(j_hbm.at[b, pl.ds(k0 + sub * kpw, kpw)],
                             jv.at[slot], sems[slot])

        def wait_slot(slot):
            pltpu.make_async_copy(r_hbm.at[0, pl.ds(0, kpw)],
                                  rv.at[slot], sems[slot]).wait()
            pltpu.make_async_copy(j_hbm.at[0, pl.ds(0, kpw)],
                                  jv.at[slot], sems[slot]).wait()

        def compute_sub(slot):
            @plsc.parallel_loop(
                0, L, step=LANES, unroll=4,
                carry=tuple(jnp.zeros((LANES,), jnp.float32)
                            for _ in range(kpw)))
            def accs(v, accs):
                off = pl.multiple_of(v, LANES)
                l_m5 = off + iota_m5
                p_i = table[pl.ds(off, LANES)]
                new = []
                for kk in range(kpw):
                    jvec = jv[slot, kk, pl.ds(off, LANES)]
                    rvec = rv[slot, kk, pl.ds(off, LANES)]
                    sep_ok = (jvec - l_m5).astype(jnp.uint32) \
                        > (2 * MIN_SEQ_SEP)
                    u = rvec * (1.0 / G_H) + G_C0
                    u = jnp.minimum(jnp.maximum(u, 0.0), G_N - 0.001)
                    idx = u.astype(jnp.int32)
                    frac = u - idx.astype(jnp.float32)
                    g = plsc.load_gather(gval_v, [idx]) \
                        + frac * plsc.load_gather(gslope_v, [idx])
                    p_j = plsc.load_gather(table, [jvec])
                    s = (p_i * p_j) * g
                    s = jnp.where(sep_ok, s, 0.0)
                    s2 = s * s
                    new.append(accs[kk] + s * s2 / (s2 + TAU_SQ))
                return tuple(new)

            total = accs[0]
            for kk in range(1, kpw):
                total = total + accs[kk]
            accv[...] = accv[...] + total

        start_sub(0, 0)
        accv[...] = jnp.zeros((LANES,), jnp.float32)
        pltpu.sync_copy(gval_hbm, gval_v)
        pltpu.sync_copy(gslope_hbm, gslope_v)
        pltpu.sync_copy(pf_hbm.at[b], table)

        def pair_body(t, _):
            sub = 2 * t
            start_sub(sub + 1, 1)
            wait_slot(0)
            compute_sub(0)

            @pl.when(sub + 2 < nsub)
            def _():
                start_sub(sub + 2, 0)

            wait_slot(1)
            compute_sub(1)
            return 0

        lax.fori_loop(0, nsub // 2, pair_body, 0)
        pltpu.sync_copy(accv, out_hbm.at[wid])

    return k(p_full, r_t, j_t, gval, gslope)


def kernel(p_ext, R, r, j_idx, lambda_raw):
    del R
    B, L, K = r.shape
    p_full = jnp.pad(p_ext, ((0, 0), (1, 0)))
    r_t = jnp.transpose(r, (0, 2, 1))
    j_t = jnp.transpose(j_idx, (0, 2, 1))
    partials = _sc_partials(p_full, r_t, j_t,
                            jnp.asarray(_GVAL), jnp.asarray(_GSLOPE),
                            B, L, K)
    e_sum = partials.reshape(B, (NW // B) * 16).sum(axis=1)
    lambda_hb = jax.nn.softplus(lambda_raw) + 1e-06
    return -lambda_hb * e_sum / float(max(L, 1))


# speedup vs baseline: 1.1053x; 1.0194x over previous
import functools

import jax
import jax.numpy as jnp
import numpy as np
from jax import lax
from jax.experimental import pallas as pl
from jax.experimental.pallas import tpu as pltpu
from jax.experimental.pallas import tpu_sc as plsc

MU1, SIGMA1, MU2, SIGMA2 = 5.79, 0.87, 10.68, 1.78
MIN_SEQ_SEP = 5
MAX_DIST = 12.0
TAU_SQ = 0.02 ** 2

NC, NS, LANES = 2, 16, 16
NW = NC * NS

G_N = 4096
G_H = 1.0 / 256.0
G_ZERO = 4088
G_RC = float(np.float32(MAX_DIST - 0.0001))
G_R0 = G_RC - G_ZERO * G_H
G_C0 = G_ZERO - G_RC / G_H


def _g_tables():
    xs = G_R0 + G_H * np.arange(G_N + 1, dtype=np.float64)
    gv = (np.exp(-0.5 * ((xs - MU1) / SIGMA1) ** 2)
          + np.exp(-0.5 * ((xs - MU2) / SIGMA2) ** 2))
    gval = gv[:-1].astype(np.float32)
    gslope = (gv[1:] - gv[:-1]).astype(np.float32)
    gval[G_ZERO:] = 0.0
    gslope[G_ZERO:] = 0.0
    return gval, gslope


_GVAL, _GSLOPE = _g_tables()


def _sc_partials(p_full, r_t, j_t, gval, gslope, B, L, K):
    halves = NW // B
    kph = K // halves
    kpw = 4
    nsub = kph // kpw

    mesh = plsc.VectorSubcoreMesh(
        core_axis_name="c", subcore_axis_name="s",
        num_cores=NC, num_subcores=NS)

    @functools.partial(
        pl.kernel,
        out_type=jax.ShapeDtypeStruct((NW, LANES), jnp.float32),
        mesh=mesh,
        compiler_params=pltpu.CompilerParams(needs_layout_passes=False),
        scratch_types=[
            pltpu.VMEM((L,), jnp.float32),
            pltpu.VMEM((2, kpw, L), jnp.float32),
            pltpu.VMEM((2, kpw, L), jnp.int32),
            pltpu.VMEM((LANES,), jnp.float32),
            pltpu.VMEM((G_N,), jnp.float32),
            pltpu.VMEM((G_N,), jnp.float32),
            pltpu.SemaphoreType.DMA,
            pltpu.SemaphoreType.DMA,
        ],
    )
    def k(pf_hbm, r_hbm, j_hbm, gval_hbm, gslope_hbm, out_hbm,
          table, rv, jv, accv, gval_v, gslope_v, sem0, sem1):
        cid = lax.axis_index("c")
        sid = lax.axis_index("s")
        wid = sid * NC + cid
        b = wid // halves
        k0 = (wid % halves) * kph
        sems = (sem0, sem1)
        iota_m5 = lax.iota(jnp.int32, LANES) - MIN_SEQ_SEP

        def start_sub(sub, slot):
            pltpu.async_copy(r_hbm.at[b, pl.ds(k0 + sub * kpw, kpw)],
                             rv.at[slot], sems[slot])
            pltpu.async_copy(j_hbm.at[b, pl.ds(k0 + sub * kpw, kpw)],
                             jv.at[slot], sems[slot])

        def wait_slot(slot):
            pltpu.make_async_copy(r_hbm.at[0, pl.ds(0, kpw)],
                                  rv.at[slot], sems[slot]).wait()
            pltpu.make_async_copy(j_hbm.at[0, pl.ds(0, kpw)],
                                  jv.at[slot], sems[slot]).wait()

        def compute_sub(slot):
            @plsc.parallel_loop(
                0, L, step=LANES, unroll=2,
                carry=tuple(jnp.zeros((LANES,), jnp.float32)
                            for _ in range(kpw)))
            def accs(v, accs):
                off = pl.multiple_of(v, LANES)
                l_m5 = off + iota_m5
                p_i = table[pl.ds(off, LANES)]
                new = []
                for kk in range(kpw):
                    jvec = jv[slot, kk, pl.ds(off, LANES)]
                    rvec = rv[slot, kk, pl.ds(off, LANES)]
                    sep_ok = (jvec - l_m5).astype(jnp.uint32) \
                        > (2 * MIN_SEQ_SEP)
                    u = rvec * (1.0 / G_H) + G_C0
                    u = jnp.minimum(jnp.maximum(u, 0.0), G_N - 0.001)
                    idx = u.astype(jnp.int32)
                    frac = u - idx.astype(jnp.float32)
                    g = plsc.load_gather(gval_v, [idx]) \
                        + frac * plsc.load_gather(gslope_v, [idx])
                    p_j = plsc.load_gather(table, [jvec])
                    s = (p_i * p_j) * g
                    s = jnp.where(sep_ok, s, 0.0)
                    s2 = s * s
                    new.append(accs[kk] + s * s2 / (s2 + TAU_SQ))
                return tuple(new)

            total = accs[0]
            for kk in range(1, kpw):
                total = total + accs[kk]
            accv[...] = accv[...] + total

        start_sub(0, 0)
        accv[...] = jnp.zeros((LANES,), jnp.float32)
        pltpu.sync_copy(gval_hbm, gval_v)
        pltpu.sync_copy(gslope_hbm, gslope_v)
        pltpu.sync_copy(pf_hbm.at[b], table)

        def pair_body(t, _):
            sub = 2 * t
            start_sub(sub + 1, 1)
            wait_slot(0)
            compute_sub(0)

            @pl.when(sub + 2 < nsub)
            def _():
                start_sub(sub + 2, 0)

            wait_slot(1)
            compute_sub(1)
            return 0

        lax.fori_loop(0, nsub // 2, pair_body, 0)
        pltpu.sync_copy(accv, out_hbm.at[wid])

    return k(p_full, r_t, j_t, gval, gslope)


def kernel(p_ext, R, r, j_idx, lambda_raw):
    del R
    B, L, K = r.shape
    p_full = jnp.pad(p_ext, ((0, 0), (1, 0)))
    r_t = jnp.transpose(r, (0, 2, 1))
    j_t = jnp.transpose(j_idx, (0, 2, 1))
    partials = _sc_partials(p_full, r_t, j_t,
                            jnp.asarray(_GVAL), jnp.asarray(_GSLOPE),
                            B, L, K)
    e_sum = partials.reshape(B, (NW // B) * 16).sum(axis=1)
    lambda_hb = jax.nn.softplus(lambda_raw) + 1e-06
    return -lambda_hb * e_sum / float(max(L, 1))
